# baseline (speedup 1.0000x reference)
import functools

import jax
import jax.numpy as jnp
from jax.experimental import pallas as pl
from jax.experimental.pallas import tpu as pltpu


def _fused_kernel(n_ff, inv_d, eps, s,
                  x_ref, wg_ref, cg_ref, w1_ref, b1_ref, w2_ref, b2_ref,
                  o_ref, gpre_ref, gpost_ref):
    j = pl.program_id(1)

    @pl.when(j == 0)
    def _gates():
        xb = x_ref[...]
        ssq = jnp.sum(xb * xb, axis=1, keepdims=True)
        r = jax.lax.rsqrt(ssq * inv_d + eps)
        t = jnp.dot(xb, wg_ref[...], preferred_element_type=jnp.float32)
        logits = r * t + cg_ref[...]
        sg = 1.0 / (1.0 + jnp.exp(-logits))
        lane = jax.lax.broadcasted_iota(jnp.int32, sg.shape, 1)
        pre_mask = (lane < s).astype(jnp.float32)
        gpre_ref[...] = jnp.sum(sg * pre_mask, axis=1, keepdims=True)
        gpost_ref[...] = 2.0 * jnp.sum(sg * (1.0 - pre_mask), axis=1,
                                       keepdims=True)
        o_ref[...] = jnp.zeros_like(o_ref)

    bi = (x_ref[...] * gpre_ref[...]).astype(jnp.bfloat16)
    h = jnp.dot(bi, w1_ref[...],
                preferred_element_type=jnp.float32)
    h = h + b1_ref[...]
    h = 0.5 * h * (1.0 + jax.lax.erf(h * 0.7071067811865476))
    o_ref[...] += jnp.dot(h.astype(jnp.bfloat16), w2_ref[...],
                          preferred_element_type=jnp.float32)

    @pl.when(j == n_ff - 1)
    def _finalize():
        o_ref[...] = (s * 1.0) * x_ref[...] + \
            (o_ref[...] + b2_ref[...]) * gpost_ref[...]


def kernel(x, phi_pre_w, phi_pre_b, phi_post_w, phi_post_b, phi_res_w,
           phi_res_b, alpha_pre, alpha_post, alpha_res, b_pre, b_post,
           b_res, W1, b1, W2, b2):
    bs, seq, d = x.shape
    s = b_pre.shape[0]
    dff = W1.shape[1]
    bt = bs * seq
    eps = float(jnp.finfo(x.dtype).eps)

    Wp = phi_pre_w.reshape(s, s, d).sum(axis=1)
    Wq = phi_post_w.reshape(s, s, d).sum(axis=1)
    Wg = jnp.concatenate([alpha_pre * Wp.T, alpha_post * Wq.T], axis=1)
    cg = jnp.concatenate([alpha_pre * phi_pre_b + b_pre,
                          alpha_post * phi_post_b + b_post]).reshape(1, 2 * s)

    x2 = x.reshape(bt, d)
    b1r = b1.reshape(1, dff)
    b2r = b2.reshape(1, d)

    tm = min(1024, bt)
    tf = min(1024, dff)
    n_tm = bt // tm
    n_ff = dff // tf
    assert n_tm * tm == bt and n_ff * tf == dff

    out = pl.pallas_call(
        functools.partial(_fused_kernel, n_ff, 1.0 / d, eps, s),
        out_shape=jax.ShapeDtypeStruct((bt, d), x.dtype),
        grid=(n_tm, n_ff),
        in_specs=[
            pl.BlockSpec((tm, d), lambda i, j: (i, 0)),
            pl.BlockSpec((d, 2 * s), lambda i, j: (0, 0)),
            pl.BlockSpec((1, 2 * s), lambda i, j: (0, 0)),
            pl.BlockSpec((d, tf), lambda i, j: (0, j)),
            pl.BlockSpec((1, tf), lambda i, j: (0, j)),
            pl.BlockSpec((tf, d), lambda i, j: (j, 0)),
            pl.BlockSpec((1, d), lambda i, j: (0, 0)),
        ],
        out_specs=pl.BlockSpec((tm, d), lambda i, j: (i, 0)),
        scratch_shapes=[
            pltpu.VMEM((tm, 1), jnp.float32),
            pltpu.VMEM((tm, 1), jnp.float32),
        ],
        compiler_params=pltpu.CompilerParams(
            dimension_semantics=("parallel", "arbitrary"),
            vmem_limit_bytes=60 * 1024 * 1024,
        ),
        name="manifold_hc_fused",
    )(x2, Wg, cg, W1.astype(jnp.bfloat16), b1r, W2.astype(jnp.bfloat16), b2r)
    return out.reshape(bs, seq, d)


# speedup vs baseline: 1.0633x; 1.0633x over previous
import functools

import jax
import jax.numpy as jnp
from jax.experimental import pallas as pl
from jax.experimental.pallas import tpu as pltpu


def _fused_kernel(n_ff, inv_d, eps, s,
                  x_ref, wg_ref, cg_ref, w1_ref, b1_ref, w2_ref, b2_ref,
                  o_ref, bi_ref, gpost_ref):
    j = pl.program_id(1)

    @pl.when(j == 0)
    def _gates():
        xb = x_ref[...]
        ssq = jnp.sum(xb * xb, axis=1, keepdims=True)
        r = jax.lax.rsqrt(ssq * inv_d + eps)
        t = jnp.dot(xb, wg_ref[...], preferred_element_type=jnp.float32)
        logits = r * t + cg_ref[...]
        sg = 1.0 / (1.0 + jnp.exp(-logits))
        lane = jax.lax.broadcasted_iota(jnp.int32, sg.shape, 1)
        pre_mask = (lane < s).astype(jnp.float32)
        g_pre = jnp.sum(sg * pre_mask, axis=1, keepdims=True)
        gpost_ref[...] = 2.0 * jnp.sum(sg * (1.0 - pre_mask), axis=1,
                                       keepdims=True)
        bi_ref[...] = (xb * g_pre).astype(jnp.bfloat16)
        o_ref[...] = jnp.zeros_like(o_ref)

    h = jnp.dot(bi_ref[...], w1_ref[...].astype(jnp.bfloat16),
                preferred_element_type=jnp.float32)
    h = h + b1_ref[...]
    h = 0.5 * h * (1.0 + jax.lax.erf(h * 0.7071067811865476))
    o_ref[...] += jnp.dot(h.astype(jnp.bfloat16),
                          w2_ref[...].astype(jnp.bfloat16),
                          preferred_element_type=jnp.float32)

    @pl.when(j == n_ff - 1)
    def _finalize():
        o_ref[...] = (s * 1.0) * x_ref[...] + \
            (o_ref[...] + b2_ref[...]) * gpost_ref[...]


def kernel(x, phi_pre_w, phi_pre_b, phi_post_w, phi_post_b, phi_res_w,
           phi_res_b, alpha_pre, alpha_post, alpha_res, b_pre, b_post,
           b_res, W1, b1, W2, b2):
    bs, seq, d = x.shape
    s = b_pre.shape[0]
    dff = W1.shape[1]
    bt = bs * seq
    eps = float(jnp.finfo(x.dtype).eps)

    Wp = phi_pre_w.reshape(s, s, d).sum(axis=1)
    Wq = phi_post_w.reshape(s, s, d).sum(axis=1)
    Wg = jnp.concatenate([alpha_pre * Wp.T, alpha_post * Wq.T], axis=1)
    cg = jnp.concatenate([alpha_pre * phi_pre_b + b_pre,
                          alpha_post * phi_post_b + b_post]).reshape(1, 2 * s)

    x2 = x.reshape(bt, d)
    b1r = b1.reshape(1, dff)
    b2r = b2.reshape(1, d)

    tm = min(1024, bt)
    tf = min(512, dff)
    n_tm = bt // tm
    n_ff = dff // tf
    assert n_tm * tm == bt and n_ff * tf == dff

    out = pl.pallas_call(
        functools.partial(_fused_kernel, n_ff, 1.0 / d, eps, s),
        out_shape=jax.ShapeDtypeStruct((bt, d), x.dtype),
        grid=(n_tm, n_ff),
        in_specs=[
            pl.BlockSpec((tm, d), lambda i, j: (i, 0)),
            pl.BlockSpec((d, 2 * s), lambda i, j: (0, 0)),
            pl.BlockSpec((1, 2 * s), lambda i, j: (0, 0)),
            pl.BlockSpec((d, tf), lambda i, j: (0, j)),
            pl.BlockSpec((1, tf), lambda i, j: (0, j)),
            pl.BlockSpec((tf, d), lambda i, j: (j, 0)),
            pl.BlockSpec((1, d), lambda i, j: (0, 0)),
        ],
        out_specs=pl.BlockSpec((tm, d), lambda i, j: (i, 0)),
        scratch_shapes=[
            pltpu.VMEM((tm, d), jnp.bfloat16),
            pltpu.VMEM((tm, 1), jnp.float32),
        ],
        compiler_params=pltpu.CompilerParams(
            dimension_semantics=("parallel", "arbitrary"),
            vmem_limit_bytes=60 * 1024 * 1024,
        ),
        name="manifold_hc_fused",
    )(x2, Wg, cg, W1, b1r, W2, b2r)
    return out.reshape(bs, seq, d)
